# baseline (speedup 1.0000x reference)
import jax
import jax.numpy as jnp
from jax import lax
from jax.experimental import pallas as pl
from jax.experimental.pallas import tpu as pltpu

B, C, H, W = 8, 768, 16, 16
N = H * W
HE, KN, KE = 64, 32, 3
COUT = 768
BPS = 8


def _tc_body(hm_ref, phi_ref, x_ref, w1_ref, b1_ref, w2_ref, b2_ref, eps_ref,
             o_ref):
    f32, bf16 = jnp.float32, jnp.bfloat16
    w1b = w1_ref[...].astype(bf16)
    w2b = w2_ref[...].astype(bf16)
    b1c = b1_ref[0][:, None]
    b2c = b2_ref[0][:, None]
    scale = (1.0 + eps_ref[0, 0]).astype(bf16)
    iota_n = lax.broadcasted_iota(jnp.int32, (N, HE), 0).astype(bf16)
    iota_e = lax.broadcasted_iota(jnp.int32, (N, HE), 1).astype(bf16)

    xbs, pns, hes = [], [], []
    for bi in range(BPS):
        xb = x_ref[bi].astype(bf16)
        hm_t = hm_ref[bi].astype(bf16).T
        phib = phi_ref[bi].astype(bf16)

        an = jnp.zeros((N, HE), bf16)
        for k in range(KN):
            row = hm_t[k, :]
            an = an + jnp.where(row[None, :] == iota_n, bf16(1.0 / KN),
                                bf16(0.0))

        pn = jnp.zeros((N, HE), bf16)
        for j in range(KE):
            col = phib[:, j]
            pn = pn + jnp.where(col[:, None] == iota_e, bf16(1.0 / KE),
                                bf16(0.0))

        xbs.append(xb)
        pns.append(pn)
        hes.append(jnp.dot(xb, an, preferred_element_type=f32).astype(bf16))

    he_cat = jnp.concatenate(hes, axis=1)
    h1_cat = jnp.maximum(
        jnp.dot(w1b, he_cat, preferred_element_type=f32) + b1c, 0.0
    ).astype(bf16)

    for bi in range(BPS):
        h1 = h1_cat[:, bi * HE:(bi + 1) * HE]
        nf = lax.dot_general(h1, pns[bi], (((1,), (1,)), ((), ())),
                             preferred_element_type=f32).astype(bf16)
        y = scale * xbs[bi] + nf
        o_ref[bi] = jnp.maximum(
            jnp.dot(w2b, y, preferred_element_type=f32) + b2c, 0.0)


def kernel(x, hyperedge_matrix, point_hyperedge_index, centers, W1, b1, W2,
           b2, eps):
    del centers
    xf = x.reshape(B, C, N)
    b1r = b1.reshape(1, C)
    b2r = b2.reshape(1, COUT)
    epsr = eps.reshape(1, 1)

    out = pl.pallas_call(
        _tc_body,
        grid=(B // BPS,),
        in_specs=[
            pl.BlockSpec((BPS, HE, KN), lambda b: (b, 0, 0)),
            pl.BlockSpec((BPS, N, KE), lambda b: (b, 0, 0)),
            pl.BlockSpec((BPS, C, N), lambda b: (b, 0, 0)),
            pl.BlockSpec((COUT, C), lambda b: (0, 0)),
            pl.BlockSpec((1, C), lambda b: (0, 0)),
            pl.BlockSpec((COUT, C), lambda b: (0, 0)),
            pl.BlockSpec((1, COUT), lambda b: (0, 0)),
            pl.BlockSpec((1, 1), lambda b: (0, 0), memory_space=pltpu.SMEM),
        ],
        out_specs=pl.BlockSpec((BPS, COUT, N), lambda b: (b, 0, 0)),
        out_shape=jax.ShapeDtypeStruct((B, COUT, N), jnp.float32),
    )(hyperedge_matrix, point_hyperedge_index, xf, W1, b1r, W2, b2r, epsr)
    return out.reshape(B, COUT, H, W)


# speedup vs baseline: 1.0216x; 1.0216x over previous
import jax
import jax.numpy as jnp
from jax import lax
from jax.experimental import pallas as pl
from jax.experimental.pallas import tpu as pltpu

B, C, H, W = 8, 768, 16, 16
N = H * W
HE, KN, KE = 64, 32, 3
COUT = 768
BPS = 4


def _tc_body(hm_ref, phi_ref, x_ref, w1_ref, b1_ref, w2_ref, b2_ref, eps_ref,
             o_ref):
    f32, bf16 = jnp.float32, jnp.bfloat16
    w1b = w1_ref[...].astype(bf16)
    w2b = w2_ref[...].astype(bf16)
    b1c = b1_ref[0][:, None]
    b2c = b2_ref[0][:, None]
    scale = (1.0 + eps_ref[0, 0]).astype(bf16)
    iota_n = lax.broadcasted_iota(jnp.int32, (N, HE), 0).astype(bf16)
    iota_e = lax.broadcasted_iota(jnp.int32, (N, HE), 1).astype(bf16)

    xbs, pns, hes = [], [], []
    for bi in range(BPS):
        xb = x_ref[bi].astype(bf16)
        hm_t = hm_ref[bi].astype(bf16).T
        phib = phi_ref[bi].astype(bf16)

        an = jnp.zeros((N, HE), bf16)
        for k in range(KN):
            row = hm_t[k, :]
            an = an + jnp.where(row[None, :] == iota_n, bf16(1.0 / KN),
                                bf16(0.0))

        pn = jnp.zeros((N, HE), bf16)
        for j in range(KE):
            col = phib[:, j]
            pn = pn + jnp.where(col[:, None] == iota_e, bf16(1.0 / KE),
                                bf16(0.0))

        xbs.append(xb)
        pns.append(pn)
        hes.append(jnp.dot(xb, an, preferred_element_type=f32).astype(bf16))

    he_cat = jnp.concatenate(hes, axis=1)
    h1_cat = jnp.maximum(
        jnp.dot(w1b, he_cat, preferred_element_type=f32) + b1c, 0.0
    ).astype(bf16)

    ys = []
    for bi in range(BPS):
        h1 = h1_cat[:, bi * HE:(bi + 1) * HE]
        nf = lax.dot_general(h1, pns[bi], (((1,), (1,)), ((), ())),
                             preferred_element_type=f32).astype(bf16)
        ys.append(scale * xbs[bi] + nf)

    y_cat = jnp.concatenate(ys, axis=1)
    out_cat = jnp.maximum(
        jnp.dot(w2b, y_cat, preferred_element_type=f32) + b2c, 0.0)
    for bi in range(BPS):
        o_ref[bi] = out_cat[:, bi * N:(bi + 1) * N]


def kernel(x, hyperedge_matrix, point_hyperedge_index, centers, W1, b1, W2,
           b2, eps):
    del centers
    xf = x.reshape(B, C, N)
    b1r = b1.reshape(1, C)
    b2r = b2.reshape(1, COUT)
    epsr = eps.reshape(1, 1)

    out = pl.pallas_call(
        _tc_body,
        grid=(B // BPS,),
        in_specs=[
            pl.BlockSpec((BPS, HE, KN), lambda b: (b, 0, 0)),
            pl.BlockSpec((BPS, N, KE), lambda b: (b, 0, 0)),
            pl.BlockSpec((BPS, C, N), lambda b: (b, 0, 0)),
            pl.BlockSpec((COUT, C), lambda b: (0, 0)),
            pl.BlockSpec((1, C), lambda b: (0, 0)),
            pl.BlockSpec((COUT, C), lambda b: (0, 0)),
            pl.BlockSpec((1, COUT), lambda b: (0, 0)),
            pl.BlockSpec((1, 1), lambda b: (0, 0), memory_space=pltpu.SMEM),
        ],
        out_specs=pl.BlockSpec((BPS, COUT, N), lambda b: (b, 0, 0)),
        out_shape=jax.ShapeDtypeStruct((B, COUT, N), jnp.float32),
    )(hyperedge_matrix, point_hyperedge_index, xf, W1, b1r, W2, b2r, epsr)
    return out.reshape(B, COUT, H, W)
